# baseline (speedup 1.0000x reference)
import functools

import jax
import jax.numpy as jnp
from jax import lax
from jax.experimental import pallas as pl
from jax.experimental.pallas import tpu as pltpu
from jax.experimental.pallas import tpu_sc as plsc

_THRESH = 0.001
_ROWS, _COLS = 128, 32768
_NC, _NS, _L = 2, 16, 16
_NW = _NC * _NS
_RPW = _ROWS // _NW

_mesh = plsc.VectorSubcoreMesh(core_axis_name="c", subcore_axis_name="s")


@functools.partial(
    pl.kernel,
    out_type=jax.ShapeDtypeStruct((_NW, _RPW, _L), jnp.float32),
    mesh=_mesh,
    compiler_params=pltpu.CompilerParams(needs_layout_passes=False),
    scratch_types=(
        pltpu.VMEM((2, _COLS), jnp.float32),
        pltpu.VMEM((_RPW, _L), jnp.float32),
        pltpu.SemaphoreType.DMA,
        pltpu.SemaphoreType.DMA,
    ),
)
def _sc_count(x_hbm, cnt_hbm, x_v, c_v, ls0, ls1):
    wid = lax.axis_index("s") * _NC + lax.axis_index("c")
    lsem = (ls0, ls1)
    loads = [None, None]
    loads[0] = pltpu.async_copy(x_hbm.at[wid * _RPW], x_v.at[0], lsem[0])
    for r in range(_RPW):
        s = r % 2
        if r + 1 < _RPW:
            loads[(r + 1) % 2] = pltpu.async_copy(
                x_hbm.at[wid * _RPW + r + 1], x_v.at[(r + 1) % 2],
                lsem[(r + 1) % 2])
        loads[s].wait()

        def body(i, a):
            v = x_v[s, pl.ds(i * _L, _L)]
            return a + jnp.where(jnp.abs(v) > _THRESH, 1.0, 0.0)

        acc = plsc.parallel_loop(
            0, _COLS // _L, 1, unroll=8,
            carry=jnp.zeros((_L,), jnp.float32))(body)
        c_v[r] = jnp.full((_L,), jnp.sum(acc), jnp.float32)
    pltpu.sync_copy(c_v, cnt_hbm.at[wid])


_BR = 8


def _tc_body(x_ref, mask_ref):
    mask_ref[...] = jnp.where(jnp.abs(x_ref[...]) > _THRESH, 1.0, 0.0)


_tc_mask = pl.pallas_call(
    _tc_body,
    grid=(_ROWS // _BR,),
    in_specs=[pl.BlockSpec((_BR, _COLS), lambda i: (i, 0))],
    out_specs=pl.BlockSpec((_BR, _COLS), lambda i: (i, 0)),
    out_shape=jax.ShapeDtypeStruct((_ROWS, _COLS), jnp.float32),
)


def kernel(x):
    cnt = _sc_count(x)
    mask = _tc_mask(x)
    sparsity = cnt[:, :, 0].reshape(_ROWS)
    return (x, sparsity, mask)


# speedup vs baseline: 1.1375x; 1.1375x over previous
import functools

import jax
import jax.numpy as jnp
from jax import lax
from jax.experimental import pallas as pl
from jax.experimental.pallas import tpu as pltpu
from jax.experimental.pallas import tpu_sc as plsc

_THRESH = 0.001
_ROWS, _COLS = 128, 32768
_NC, _NS, _L = 2, 16, 16
_NW = _NC * _NS
_RPW = _ROWS // _NW

_mesh = plsc.VectorSubcoreMesh(core_axis_name="c", subcore_axis_name="s")


@functools.partial(
    pl.kernel,
    out_type=(
        jax.ShapeDtypeStruct((_ROWS, _COLS), jnp.float32),
        jax.ShapeDtypeStruct((_NW, _RPW, _L), jnp.float32),
    ),
    mesh=_mesh,
    compiler_params=pltpu.CompilerParams(needs_layout_passes=False),
    scratch_types=(
        pltpu.VMEM((2, _COLS), jnp.float32),
        pltpu.VMEM((_RPW, _L), jnp.float32),
        pltpu.SemaphoreType.DMA,
        pltpu.SemaphoreType.DMA,
        pltpu.SemaphoreType.DMA,
        pltpu.SemaphoreType.DMA,
    ),
)
def _sc_count_copy(x_hbm, xs_hbm, cnt_hbm, x_v, c_v, ls0, ls1, ss0, ss1):
    wid = lax.axis_index("s") * _NC + lax.axis_index("c")
    lsem = (ls0, ls1)
    ssem = (ss0, ss1)
    loads = [None, None]
    stores = [None, None]
    loads[0] = pltpu.async_copy(x_hbm.at[wid * _RPW], x_v.at[0], lsem[0])
    for r in range(_RPW):
        s = r % 2
        if r + 1 < _RPW:
            ns = (r + 1) % 2
            if stores[ns] is not None:
                stores[ns].wait()
            loads[ns] = pltpu.async_copy(
                x_hbm.at[wid * _RPW + r + 1], x_v.at[ns], lsem[ns])
        loads[s].wait()

        def body(i, a):
            v = x_v[s, pl.ds(i * _L, _L)]
            return a + jnp.where(jnp.abs(v) > _THRESH, 1.0, 0.0)

        acc = plsc.parallel_loop(
            0, _COLS // _L, 1, unroll=8,
            carry=jnp.zeros((_L,), jnp.float32))(body)
        c_v[r] = jnp.full((_L,), jnp.sum(acc), jnp.float32)
        stores[s] = pltpu.async_copy(x_v.at[s], xs_hbm.at[wid * _RPW + r],
                                     ssem[s])
    stores[0].wait()
    stores[1].wait()
    pltpu.sync_copy(c_v, cnt_hbm.at[wid])


_BR = 8


def _tc_body(x_ref, mask_ref):
    mask_ref[...] = jnp.where(jnp.abs(x_ref[...]) > _THRESH, 1.0, 0.0)


_tc_mask = pl.pallas_call(
    _tc_body,
    grid=(_ROWS // _BR,),
    in_specs=[pl.BlockSpec((_BR, _COLS), lambda i: (i, 0))],
    out_specs=pl.BlockSpec((_BR, _COLS), lambda i: (i, 0)),
    out_shape=jax.ShapeDtypeStruct((_ROWS, _COLS), jnp.float32),
)


def kernel(x):
    x_sparse, cnt = _sc_count_copy(x)
    mask = _tc_mask(x)
    sparsity = cnt[:, :, 0].reshape(_ROWS)
    return (x_sparse, sparsity, mask)


# speedup vs baseline: 1.1892x; 1.0454x over previous
import functools

import jax
import jax.numpy as jnp
from jax import lax
from jax.experimental import pallas as pl
from jax.experimental.pallas import tpu as pltpu
from jax.experimental.pallas import tpu_sc as plsc

_THRESH = 0.001
_ROWS, _COLS = 128, 32768
_NC, _NS, _L = 2, 16, 16
_NW = _NC * _NS
_RPW = _ROWS // _NW

_mesh = plsc.VectorSubcoreMesh(core_axis_name="c", subcore_axis_name="s")


@functools.partial(
    pl.kernel,
    out_type=jax.ShapeDtypeStruct((_NW, _RPW, _L), jnp.float32),
    mesh=_mesh,
    compiler_params=pltpu.CompilerParams(needs_layout_passes=False),
    scratch_types=(
        pltpu.VMEM((2, _COLS), jnp.float32),
        pltpu.VMEM((_RPW, _L), jnp.float32),
        pltpu.SemaphoreType.DMA,
        pltpu.SemaphoreType.DMA,
    ),
)
def _sc_count(x_hbm, cnt_hbm, x_v, c_v, ls0, ls1):
    wid = lax.axis_index("s") * _NC + lax.axis_index("c")
    lsem = (ls0, ls1)
    loads = [None, None]
    loads[0] = pltpu.async_copy(x_hbm.at[wid * _RPW], x_v.at[0], lsem[0])
    for r in range(_RPW):
        s = r % 2
        if r + 1 < _RPW:
            loads[(r + 1) % 2] = pltpu.async_copy(
                x_hbm.at[wid * _RPW + r + 1], x_v.at[(r + 1) % 2],
                lsem[(r + 1) % 2])
        loads[s].wait()

        def body(i, a):
            v = x_v[s, pl.ds(i * _L, _L)]
            m = jnp.abs(v) > _THRESH
            return a + plsc.all_reduce_population_count(m)

        acc = plsc.parallel_loop(
            0, _COLS // _L, 1, unroll=8,
            carry=jnp.zeros((_L,), jnp.int32))(body)
        c_v[r] = acc.astype(jnp.float32)
    pltpu.sync_copy(c_v, cnt_hbm.at[wid])


_BR = 8


def _tc_body(x_ref, copy_ref, mask_ref):
    v = x_ref[...]
    copy_ref[...] = v
    mask_ref[...] = jnp.where(jnp.abs(v) > _THRESH, 1.0, 0.0)


_tc_mask_copy = pl.pallas_call(
    _tc_body,
    grid=(_ROWS // _BR,),
    in_specs=[pl.BlockSpec((_BR, _COLS), lambda i: (i, 0))],
    out_specs=[
        pl.BlockSpec((_BR, _COLS), lambda i: (i, 0)),
        pl.BlockSpec((_BR, _COLS), lambda i: (i, 0)),
    ],
    out_shape=[
        jax.ShapeDtypeStruct((_ROWS, _COLS), jnp.float32),
        jax.ShapeDtypeStruct((_ROWS, _COLS), jnp.float32),
    ],
)


def kernel(x):
    cnt = _sc_count(x)
    x_sparse, mask = _tc_mask_copy(x)
    sparsity = cnt[:, :, 0].reshape(_ROWS)
    return (x_sparse, sparsity, mask)
